# Initial kernel scaffold; baseline (speedup 1.0000x reference)
import jax
import jax.numpy as jnp
from jax.experimental import pallas as pl


def kernel(x, edge_index, Wq, bq, Wk, bk, Wv, bv, Wo, bo, gamma, beta):
    raise NotImplementedError("write your pallas kernel here")


# speedup vs baseline: 7.6058x; 7.6058x over previous
import functools

import numpy as np

import jax
import jax.numpy as jnp
from jax import lax
from jax.experimental import pallas as pl
from jax.experimental.pallas import tpu as pltpu
from jax.experimental.pallas import tpu_sc as plsc

_N = 10000
_E = 160000
_D = 256
_H = 8
_DH = 32
_EPS = 1e-12

_C = 64
_NS = 16
_NC = 2
_CHUNKS = _E // _C
_BASE_K = _CHUNKS // _NS
_EXTRA_TILES = _CHUNKS % _NS
_NP = 10240
_RPT = _NP // _NS
_CW = 144


def _sc_edge(src2, dst2, dstl, qst, kst, vst, acc_hbm,
             sidx, gdidx, didx, krows, qrows, vrows, srow, gsem, acc_sh):
    c = lax.axis_index("c")
    s = lax.axis_index("s")
    iota16 = lax.iota(jnp.int32, 16)
    zero16 = jnp.zeros((16,), jnp.float32)

    @pl.loop(0, _C)
    def _zero(i):
        for j in range(_CW // 16):
            srow[i, pl.ds(j * 16, 16)] = zero16

    rowbase = s * _RPT
    for t in range(_RPT // _C):
        pltpu.sync_copy(srow, acc_sh.at[pl.ds(rowbase + t * _C, _C)])
    plsc.subcore_barrier()

    nk = jnp.where(s < _EXTRA_TILES, _BASE_K + 1, _BASE_K)

    @pl.loop(0, nk)
    def _chunk(k):
        ebase = (k * _NS + s) * _C
        gbase = ebase + c * _E
        pltpu.sync_copy(src2.at[pl.ds(gbase, _C)], sidx)
        pltpu.sync_copy(dst2.at[pl.ds(gbase, _C)], gdidx)
        pltpu.sync_copy(dstl.at[pl.ds(ebase, _C)], didx)
        d1 = pltpu.async_copy(kst.at[sidx], krows, gsem)
        d2 = pltpu.async_copy(qst.at[gdidx], qrows, gsem)
        d3 = pltpu.async_copy(vst.at[sidx], vrows, gsem)
        d1.wait()
        d2.wait()
        d3.wait()

        @pl.loop(0, _C // 16)
        def _grp(g):
            e16 = g * 16 + iota16

            @pl.loop(0, _H // _NC)
            def _head(h):
                hbase = h * _DH
                acc = zero16
                for d in range(_DH):
                    col = jnp.full((16,), hbase + d, jnp.int32)
                    kk = plsc.load_gather(krows, [e16, col])
                    qq = plsc.load_gather(qrows, [e16, col])
                    acc = acc + kk * qq
                ex = jnp.exp(acc)
                plsc.store_scatter(srow, [e16, jnp.full((16,), 128 + h, jnp.int32)], ex)
                for d in range(_DH):
                    col = jnp.full((16,), hbase + d, jnp.int32)
                    vv = plsc.load_gather(vrows, [e16, col])
                    plsc.store_scatter(srow, [e16, col], vv * ex)

        pltpu.sync_copy(srow, acc_sh.at[didx], add=True)

    plsc.subcore_barrier()

    @pl.loop(0, _RPT // _C)
    def _dump(t):
        rb = s * _RPT + t * _C
        ob = c * _NP + rb
        pltpu.sync_copy(acc_sh.at[pl.ds(rb, _C)], srow)
        pltpu.sync_copy(srow, acc_hbm.at[pl.ds(ob, _C)])


@functools.cache
def _make_edge_call():
  return pl.kernel(
    _sc_edge,
    out_type=jax.ShapeDtypeStruct((_NC * _NP, _CW), jnp.float32),
    mesh=plsc.VectorSubcoreMesh(core_axis_name="c", subcore_axis_name="s",
                                num_cores=_NC, num_subcores=_NS),
    compiler_params=pltpu.CompilerParams(needs_layout_passes=False,
                                         use_tc_tiling_on_sc=False),
    scratch_types=[
        pltpu.VMEM((_C,), jnp.int32),
        pltpu.VMEM((_C,), jnp.int32),
        pltpu.VMEM((_C,), jnp.int32),
        pltpu.VMEM((_C, 128), jnp.float32),
        pltpu.VMEM((_C, 128), jnp.float32),
        pltpu.VMEM((_C, 128), jnp.float32),
        pltpu.VMEM((_C, _CW), jnp.float32),
        pltpu.SemaphoreType.DMA,
        pltpu.VMEM_SHARED((_NP, _CW), jnp.float32),
    ],
  )

_BROW = 400


def _proj_body(x_ref, w_ref, b_ref, q_ref, k_ref, v_ref):
    y = jnp.dot(x_ref[...], w_ref[...],
                preferred_element_type=jnp.float32) + b_ref[...]
    q_ref[0] = y[:, 0:128]
    q_ref[1] = y[:, 128:256]
    k_ref[0] = y[:, 256:384]
    k_ref[1] = y[:, 384:512]
    v_ref[0] = y[:, 512:640]
    v_ref[1] = y[:, 640:768]


_proj_call = pl.pallas_call(
    _proj_body,
    grid=(_N // _BROW,),
    in_specs=[
        pl.BlockSpec((_BROW, _D), lambda i: (i, 0)),
        pl.BlockSpec((_D, 3 * _D), lambda i: (0, 0)),
        pl.BlockSpec((1, 3 * _D), lambda i: (0, 0)),
    ],
    out_specs=[
        pl.BlockSpec((_NC, _BROW, 128), lambda i: (0, i, 0)),
        pl.BlockSpec((_NC, _BROW, 128), lambda i: (0, i, 0)),
        pl.BlockSpec((_NC, _BROW, 128), lambda i: (0, i, 0)),
    ],
    out_shape=[jax.ShapeDtypeStruct((_NC, _N, 128), jnp.float32)] * 3,
)


def _out_body(acc_ref, x_ref, wo_ref, bo_ref, g_ref, bt_ref, er_ref, o_ref):
    hh = jnp.concatenate([acc_ref[0][:, 0:128], acc_ref[1][:, 0:128]], axis=1)
    dd = jnp.concatenate([acc_ref[0][:, 128:132], acc_ref[1][:, 128:132]],
                         axis=1)
    rec = 1.0 / jnp.maximum(dd, 1e-30)
    scale = jnp.dot(rec, er_ref[...], preferred_element_type=jnp.float32)
    h = hh * scale
    out = jnp.dot(h, wo_ref[...], preferred_element_type=jnp.float32)
    out = out + bo_ref[...] + x_ref[...]
    mu = jnp.mean(out, axis=-1, keepdims=True)
    zz = out - mu
    var = jnp.mean(zz * zz, axis=-1, keepdims=True)
    o_ref[...] = zz * lax.rsqrt(var + _EPS) * g_ref[...] + bt_ref[...]


_out_call = pl.pallas_call(
    _out_body,
    grid=(_N // _BROW,),
    in_specs=[
        pl.BlockSpec((_NC, _BROW, _CW), lambda i: (0, i, 0)),
        pl.BlockSpec((_BROW, _D), lambda i: (i, 0)),
        pl.BlockSpec((_D, _D), lambda i: (0, 0)),
        pl.BlockSpec((1, _D), lambda i: (0, 0)),
        pl.BlockSpec((1, _D), lambda i: (0, 0)),
        pl.BlockSpec((1, _D), lambda i: (0, 0)),
        pl.BlockSpec((_H, _D), lambda i: (0, 0)),
    ],
    out_specs=pl.BlockSpec((_BROW, _D), lambda i: (i, 0)),
    out_shape=jax.ShapeDtypeStruct((_N, _D), jnp.float32),
)

_EREP = np.repeat(np.eye(_H, dtype=np.float32), _DH, axis=1)


def kernel(x, edge_index, Wq, bq, Wk, bk, Wv, bv, Wo, bo, gamma, beta):
    src = edge_index[0]
    dst = edge_index[1]
    Wqkv = jnp.concatenate([Wq, Wk, Wv], axis=1)
    bqkv = jnp.concatenate([bq, bk, bv]).reshape(1, 3 * _D)
    qst, kst, vst = _proj_call(x, Wqkv, bqkv)
    src2 = jnp.concatenate([src, src + _N])
    dst2 = jnp.concatenate([dst, dst + _N])
    acc = _make_edge_call()(
        src2, dst2, dst,
        qst.reshape(_NC * _N, 128),
        kst.reshape(_NC * _N, 128),
        vst.reshape(_NC * _N, 128),
    )
    return _out_call(
        acc.reshape(_NC, _NP, _CW),
        x, Wo, bo.reshape(1, _D), gamma.reshape(1, _D), beta.reshape(1, _D),
        _EREP,
    )


# speedup vs baseline: 8.3079x; 1.0923x over previous
import functools

import numpy as np

import jax
import jax.numpy as jnp
from jax import lax
from jax.experimental import pallas as pl
from jax.experimental.pallas import tpu as pltpu
from jax.experimental.pallas import tpu_sc as plsc

_N = 10000
_E = 160000
_D = 256
_H = 8
_DH = 32
_EPS = 1e-12

_C = 64
_NS = 16
_NC = 2
_CHUNKS = _E // _C
_BASE_K = _CHUNKS // _NS
_EXTRA_TILES = _CHUNKS % _NS
_NP = 10240
_RPT = _NP // _NS
_CW = 144


_SB = 16
_CP = 2512
_NSB = (_BASE_K + 1 + _SB - 1) // _SB
_SROW_BYTES = _C * _CW * 4


def _sc_edge(src3, dst3, dstl3, qst, kst, vst, acc_hbm,
             sidxb, gdidxb, didxb, krows, qrows, vrows, srow,
             gsem, ssem, acc_sh):
    c = lax.axis_index("c")
    s = lax.axis_index("s")
    iota16 = lax.iota(jnp.int32, 16)
    zero16 = jnp.zeros((16,), jnp.float32)

    @pl.loop(0, _C)
    def _zero(i):
        for j in range(_CW // 16):
            srow[i, pl.ds(j * 16, 16)] = zero16

    rowbase = s * _RPT
    for t in range(_RPT // _C):
        pltpu.sync_copy(srow, acc_sh.at[pl.ds(rowbase + t * _C, _C)])
    plsc.subcore_barrier()

    sb0 = s * _BASE_K + jnp.minimum(s, _EXTRA_TILES)
    nk = jnp.where(s < _EXTRA_TILES, _BASE_K + 1, _BASE_K)

    @pl.loop(0, _NSB)
    def _sblock(t):
        @pl.when(t > 0)
        def _():
            pltpu.make_async_copy(acc_hbm.at[pl.ds(0, _C)], srow, ssem).wait()
        base = sb0 + t * _SB
        pltpu.sync_copy(src3.at[c, pl.ds(base, _SB)], sidxb)
        pltpu.sync_copy(dst3.at[c, pl.ds(base, _SB)], gdidxb)
        pltpu.sync_copy(dstl3.at[pl.ds(base, _SB)], didxb)
        jmax = jnp.minimum(_SB, nk - t * _SB)

        @pl.loop(0, jmax)
        def _chunk(j):
            d1 = pltpu.async_copy(kst.at[sidxb.at[j]], krows, gsem)
            d2 = pltpu.async_copy(qst.at[gdidxb.at[j]], qrows, gsem)
            d3 = pltpu.async_copy(vst.at[sidxb.at[j]], vrows, gsem)
            d1.wait()
            d2.wait()
            d3.wait()

            @pl.when(j > 0)
            def _():
                pltpu.make_async_copy(acc_hbm.at[pl.ds(0, _C)], srow, ssem).wait()

            @pl.loop(0, _C // 16)
            def _grp(g):
                e16 = g * 16 + iota16

                @pl.loop(0, _H // _NC)
                def _head(h):
                    hbase = h * _DH
                    acc = zero16
                    for d in range(_DH):
                        col = jnp.full((16,), hbase + d, jnp.int32)
                        kk = plsc.load_gather(krows, [e16, col])
                        qq = plsc.load_gather(qrows, [e16, col])
                        acc = acc + kk * qq
                    ex = jnp.exp(acc)
                    plsc.store_scatter(
                        srow, [e16, jnp.full((16,), 128 + h, jnp.int32)], ex)
                    for d in range(_DH):
                        col = jnp.full((16,), hbase + d, jnp.int32)
                        vv = plsc.load_gather(vrows, [e16, col])
                        plsc.store_scatter(srow, [e16, col], vv * ex)

            pltpu.async_copy(srow, acc_sh.at[didxb.at[j]], ssem, add=True)

    pltpu.make_async_copy(acc_hbm.at[pl.ds(0, _C)], srow, ssem).wait()
    plsc.subcore_barrier()

    @pl.loop(0, _RPT // _C)
    def _dump(t):
        rb = s * _RPT + t * _C
        ob = c * _NP + rb
        pltpu.sync_copy(acc_sh.at[pl.ds(rb, _C)], srow)
        pltpu.sync_copy(srow, acc_hbm.at[pl.ds(ob, _C)])


@functools.cache
def _make_edge_call():
  return pl.kernel(
    _sc_edge,
    out_type=jax.ShapeDtypeStruct((_NC * _NP, _CW), jnp.float32),
    mesh=plsc.VectorSubcoreMesh(core_axis_name="c", subcore_axis_name="s",
                                num_cores=_NC, num_subcores=_NS),
    compiler_params=pltpu.CompilerParams(needs_layout_passes=False,
                                         use_tc_tiling_on_sc=False),
    scratch_types=[
        pltpu.VMEM((_SB, _C), jnp.int32),
        pltpu.VMEM((_SB, _C), jnp.int32),
        pltpu.VMEM((_SB, _C), jnp.int32),
        pltpu.VMEM((_C, 128), jnp.float32),
        pltpu.VMEM((_C, 128), jnp.float32),
        pltpu.VMEM((_C, 128), jnp.float32),
        pltpu.VMEM((_C, _CW), jnp.float32),
        pltpu.SemaphoreType.DMA,
        pltpu.SemaphoreType.DMA,
        pltpu.VMEM_SHARED((_NP, _CW), jnp.float32),
    ],
  )

_BROW = 400


def _proj_body(x_ref, w_ref, b_ref, q_ref, k_ref, v_ref):
    y = jnp.dot(x_ref[...], w_ref[...],
                preferred_element_type=jnp.float32) + b_ref[...]
    q_ref[0] = y[:, 0:128]
    q_ref[1] = y[:, 128:256]
    k_ref[0] = y[:, 256:384]
    k_ref[1] = y[:, 384:512]
    v_ref[0] = y[:, 512:640]
    v_ref[1] = y[:, 640:768]


_proj_call = pl.pallas_call(
    _proj_body,
    grid=(_N // _BROW,),
    in_specs=[
        pl.BlockSpec((_BROW, _D), lambda i: (i, 0)),
        pl.BlockSpec((_D, 3 * _D), lambda i: (0, 0)),
        pl.BlockSpec((1, 3 * _D), lambda i: (0, 0)),
    ],
    out_specs=[
        pl.BlockSpec((_NC, _BROW, 128), lambda i: (0, i, 0)),
        pl.BlockSpec((_NC, _BROW, 128), lambda i: (0, i, 0)),
        pl.BlockSpec((_NC, _BROW, 128), lambda i: (0, i, 0)),
    ],
    out_shape=[jax.ShapeDtypeStruct((_NC, _N, 128), jnp.float32)] * 3,
)


def _out_body(acc_ref, x_ref, wo_ref, bo_ref, g_ref, bt_ref, er_ref, o_ref):
    hh = jnp.concatenate([acc_ref[0][:, 0:128], acc_ref[1][:, 0:128]], axis=1)
    dd = jnp.concatenate([acc_ref[0][:, 128:132], acc_ref[1][:, 128:132]],
                         axis=1)
    rec = 1.0 / jnp.maximum(dd, 1e-30)
    scale = jnp.dot(rec, er_ref[...], preferred_element_type=jnp.float32)
    h = hh * scale
    out = jnp.dot(h, wo_ref[...], preferred_element_type=jnp.float32)
    out = out + bo_ref[...] + x_ref[...]
    mu = jnp.mean(out, axis=-1, keepdims=True)
    zz = out - mu
    var = jnp.mean(zz * zz, axis=-1, keepdims=True)
    o_ref[...] = zz * lax.rsqrt(var + _EPS) * g_ref[...] + bt_ref[...]


_out_call = pl.pallas_call(
    _out_body,
    grid=(_N // _BROW,),
    in_specs=[
        pl.BlockSpec((_NC, _BROW, _CW), lambda i: (0, i, 0)),
        pl.BlockSpec((_BROW, _D), lambda i: (i, 0)),
        pl.BlockSpec((_D, _D), lambda i: (0, 0)),
        pl.BlockSpec((1, _D), lambda i: (0, 0)),
        pl.BlockSpec((1, _D), lambda i: (0, 0)),
        pl.BlockSpec((1, _D), lambda i: (0, 0)),
        pl.BlockSpec((_H, _D), lambda i: (0, 0)),
    ],
    out_specs=pl.BlockSpec((_BROW, _D), lambda i: (i, 0)),
    out_shape=jax.ShapeDtypeStruct((_N, _D), jnp.float32),
)

_EREP = np.repeat(np.eye(_H, dtype=np.float32), _DH, axis=1)


def kernel(x, edge_index, Wq, bq, Wk, bk, Wv, bv, Wo, bo, gamma, beta):
    src = edge_index[0]
    dst = edge_index[1]
    Wqkv = jnp.concatenate([Wq, Wk, Wv], axis=1)
    bqkv = jnp.concatenate([bq, bk, bv]).reshape(1, 3 * _D)
    qst, kst, vst = _proj_call(x, Wqkv, bqkv)
    pad = jnp.zeros((_CP * _C - _E,), jnp.int32)
    srcp = jnp.concatenate([src, pad]).reshape(_CP, _C)
    dstp = jnp.concatenate([dst, pad]).reshape(_CP, _C)
    src3 = jnp.stack([srcp, srcp + _N])
    dst3 = jnp.stack([dstp, dstp + _N])
    acc = _make_edge_call()(
        src3, dst3, dstp,
        qst.reshape(_NC * _N, 128),
        kst.reshape(_NC * _N, 128),
        vst.reshape(_NC * _N, 128),
    )
    return _out_call(
        acc.reshape(_NC, _NP, _CW),
        x, Wo, bo.reshape(1, _D), gamma.reshape(1, _D), beta.reshape(1, _D),
        _EREP,
    )


# speedup vs baseline: 8.6942x; 1.0465x over previous
import functools

import numpy as np

import jax
import jax.numpy as jnp
from jax import lax
from jax.experimental import pallas as pl
from jax.experimental.pallas import tpu as pltpu
from jax.experimental.pallas import tpu_sc as plsc

_N = 10000
_E = 160000
_D = 256
_H = 8
_DH = 32
_EPS = 1e-12

_C = 32
_NS = 16
_NC = 2
_CHUNKS = _E // _C
_BASE_K = _CHUNKS // _NS
_EXTRA_TILES = _CHUNKS % _NS
_NP = 10240
_RPT = _NP // _NS
_CW = 144


_SB = 16
_CP = 5008
_NSB = (_BASE_K + 1 + _SB - 1) // _SB


def _sc_edge(src3, dst3, dstl3, qst, kst, vst, acc_hbm,
             sidxb, gdidxb, didxb, krows, qrows, vrows, srow,
             gsem, ssem, acc_sh):
    c = lax.axis_index("c")
    s = lax.axis_index("s")
    iota16 = lax.iota(jnp.int32, 16)
    zero16 = jnp.zeros((16,), jnp.float32)

    @pl.loop(0, _C)
    def _zero(i):
        for j in range(_CW // 16):
            srow[i, pl.ds(j * 16, 16)] = zero16

    rowbase = s * _RPT
    for t in range(_RPT // _C):
        pltpu.sync_copy(srow, acc_sh.at[pl.ds(rowbase + t * _C, _C)])
    plsc.subcore_barrier()

    sb0 = s * _BASE_K + jnp.minimum(s, _EXTRA_TILES)
    nk = jnp.where(s < _EXTRA_TILES, _BASE_K + 1, _BASE_K)

    def _issue(j, bb):
        off = bb * _C
        pltpu.async_copy(kst.at[sidxb.at[j]], krows.at[pl.ds(off, _C)],
                         gsem.at[bb])
        pltpu.async_copy(qst.at[gdidxb.at[j]], qrows.at[pl.ds(off, _C)],
                         gsem.at[bb])
        pltpu.async_copy(vst.at[sidxb.at[j]], vrows.at[pl.ds(off, _C)],
                         gsem.at[bb])

    def _drain_g(bb):
        off = bb * _C
        dummy = kst.at[pl.ds(0, _C)]
        pltpu.make_async_copy(dummy, krows.at[pl.ds(off, _C)], gsem.at[bb]).wait()
        pltpu.make_async_copy(dummy, qrows.at[pl.ds(off, _C)], gsem.at[bb]).wait()
        pltpu.make_async_copy(dummy, vrows.at[pl.ds(off, _C)], gsem.at[bb]).wait()

    def _drain_s():
        pltpu.make_async_copy(acc_hbm.at[pl.ds(0, _C)], srow, ssem).wait()

    @pl.loop(0, _NSB)
    def _sblock(t):
        @pl.when(t > 0)
        def _():
            _drain_s()
        base = sb0 + t * _SB
        pltpu.sync_copy(src3.at[c, pl.ds(base, _SB)], sidxb)
        pltpu.sync_copy(dst3.at[c, pl.ds(base, _SB)], gdidxb)
        pltpu.sync_copy(dstl3.at[pl.ds(base, _SB)], didxb)
        jmax = jnp.minimum(_SB, nk - t * _SB)
        _issue(0, 0)

        @pl.loop(0, jmax)
        def _chunk(j):
            bb = j & 1

            @pl.when(j + 1 < jmax)
            def _():
                _issue(j + 1, 1 - bb)

            _drain_g(bb)

            @pl.when(j > 0)
            def _():
                _drain_s()

            eoff = bb * _C

            @pl.loop(0, _C // 16)
            def _grp(g):
                e16l = g * 16 + iota16
                e16 = eoff + e16l

                @pl.loop(0, _H // _NC)
                def _head(h):
                    hbase = h * _DH
                    acc = zero16
                    for d in range(_DH):
                        col = jnp.full((16,), hbase + d, jnp.int32)
                        kk = plsc.load_gather(krows, [e16, col])
                        qq = plsc.load_gather(qrows, [e16, col])
                        acc = acc + kk * qq
                    ex = jnp.exp(acc)
                    plsc.store_scatter(
                        srow, [e16l, jnp.full((16,), 128 + h, jnp.int32)], ex)
                    for d in range(_DH):
                        col = jnp.full((16,), hbase + d, jnp.int32)
                        vv = plsc.load_gather(vrows, [e16, col])
                        plsc.store_scatter(srow, [e16l, col], vv * ex)

            pltpu.async_copy(srow, acc_sh.at[didxb.at[j]], ssem, add=True)

    _drain_s()
    plsc.subcore_barrier()

    @pl.loop(0, _RPT // _C)
    def _dump(t):
        rb = s * _RPT + t * _C
        ob = c * _NP + rb
        pltpu.sync_copy(acc_sh.at[pl.ds(rb, _C)], srow)
        pltpu.sync_copy(srow, acc_hbm.at[pl.ds(ob, _C)])


@functools.cache
def _make_edge_call():
  return pl.kernel(
    _sc_edge,
    out_type=jax.ShapeDtypeStruct((_NC * _NP, _CW), jnp.float32),
    mesh=plsc.VectorSubcoreMesh(core_axis_name="c", subcore_axis_name="s",
                                num_cores=_NC, num_subcores=_NS),
    compiler_params=pltpu.CompilerParams(needs_layout_passes=False,
                                         use_tc_tiling_on_sc=False),
    scratch_types=[
        pltpu.VMEM((_SB, _C), jnp.int32),
        pltpu.VMEM((_SB, _C), jnp.int32),
        pltpu.VMEM((_SB, _C), jnp.int32),
        pltpu.VMEM((2 * _C, 128), jnp.float32),
        pltpu.VMEM((2 * _C, 128), jnp.float32),
        pltpu.VMEM((2 * _C, 128), jnp.float32),
        pltpu.VMEM((_C, _CW), jnp.float32),
        pltpu.SemaphoreType.DMA((2,)),
        pltpu.SemaphoreType.DMA,
        pltpu.VMEM_SHARED((_NP, _CW), jnp.float32),
    ],
  )

_BROW = 400


def _proj_body(x_ref, w_ref, b_ref, q_ref, k_ref, v_ref):
    y = jnp.dot(x_ref[...], w_ref[...],
                preferred_element_type=jnp.float32) + b_ref[...]
    q_ref[0] = y[:, 0:128]
    q_ref[1] = y[:, 128:256]
    k_ref[0] = y[:, 256:384]
    k_ref[1] = y[:, 384:512]
    v_ref[0] = y[:, 512:640]
    v_ref[1] = y[:, 640:768]


_proj_call = pl.pallas_call(
    _proj_body,
    grid=(_N // _BROW,),
    in_specs=[
        pl.BlockSpec((_BROW, _D), lambda i: (i, 0)),
        pl.BlockSpec((_D, 3 * _D), lambda i: (0, 0)),
        pl.BlockSpec((1, 3 * _D), lambda i: (0, 0)),
    ],
    out_specs=[
        pl.BlockSpec((_NC, _BROW, 128), lambda i: (0, i, 0)),
        pl.BlockSpec((_NC, _BROW, 128), lambda i: (0, i, 0)),
        pl.BlockSpec((_NC, _BROW, 128), lambda i: (0, i, 0)),
    ],
    out_shape=[jax.ShapeDtypeStruct((_NC, _N, 128), jnp.float32)] * 3,
)


def _out_body(acc_ref, x_ref, wo_ref, bo_ref, g_ref, bt_ref, er_ref, o_ref):
    hh = jnp.concatenate([acc_ref[0][:, 0:128], acc_ref[1][:, 0:128]], axis=1)
    dd = jnp.concatenate([acc_ref[0][:, 128:132], acc_ref[1][:, 128:132]],
                         axis=1)
    rec = 1.0 / jnp.maximum(dd, 1e-30)
    scale = jnp.dot(rec, er_ref[...], preferred_element_type=jnp.float32)
    h = hh * scale
    out = jnp.dot(h, wo_ref[...], preferred_element_type=jnp.float32)
    out = out + bo_ref[...] + x_ref[...]
    mu = jnp.mean(out, axis=-1, keepdims=True)
    zz = out - mu
    var = jnp.mean(zz * zz, axis=-1, keepdims=True)
    o_ref[...] = zz * lax.rsqrt(var + _EPS) * g_ref[...] + bt_ref[...]


_out_call = pl.pallas_call(
    _out_body,
    grid=(_N // _BROW,),
    in_specs=[
        pl.BlockSpec((_NC, _BROW, _CW), lambda i: (0, i, 0)),
        pl.BlockSpec((_BROW, _D), lambda i: (i, 0)),
        pl.BlockSpec((_D, _D), lambda i: (0, 0)),
        pl.BlockSpec((1, _D), lambda i: (0, 0)),
        pl.BlockSpec((1, _D), lambda i: (0, 0)),
        pl.BlockSpec((1, _D), lambda i: (0, 0)),
        pl.BlockSpec((_H, _D), lambda i: (0, 0)),
    ],
    out_specs=pl.BlockSpec((_BROW, _D), lambda i: (i, 0)),
    out_shape=jax.ShapeDtypeStruct((_N, _D), jnp.float32),
)

_EREP = np.repeat(np.eye(_H, dtype=np.float32), _DH, axis=1)


def kernel(x, edge_index, Wq, bq, Wk, bk, Wv, bv, Wo, bo, gamma, beta):
    src = edge_index[0]
    dst = edge_index[1]
    Wqkv = jnp.concatenate([Wq, Wk, Wv], axis=1)
    bqkv = jnp.concatenate([bq, bk, bv]).reshape(1, 3 * _D)
    qst, kst, vst = _proj_call(x, Wqkv, bqkv)
    pad = jnp.zeros((_CP * _C - _E,), jnp.int32)
    srcp = jnp.concatenate([src, pad]).reshape(_CP, _C)
    dstp = jnp.concatenate([dst, pad]).reshape(_CP, _C)
    src3 = jnp.stack([srcp, srcp + _N])
    dst3 = jnp.stack([dstp, dstp + _N])
    acc = _make_edge_call()(
        src3, dst3, dstp,
        qst.reshape(_NC * _N, 128),
        kst.reshape(_NC * _N, 128),
        vst.reshape(_NC * _N, 128),
    )
    return _out_call(
        acc.reshape(_NC, _NP, _CW),
        x, Wo, bo.reshape(1, _D), gamma.reshape(1, _D), beta.reshape(1, _D),
        _EREP,
    )
